# baseline (speedup 1.0000x reference)
import jax, jax.numpy as jnp
import math
from jax.experimental import pallas as pl

_B, _N, _K = 8, 2048, 32
_CIN, _COUT, _H = 3, 64, 8
_DEPTH = _COUT // _H


def _identity_kernel(x_ref, o_ref):
    o_ref[...] = x_ref[...]


def kernel(x, coordinate, Wq, Wk, Wv):
    ct = coordinate.transpose(0, 2, 1)
    sq = jnp.sum(ct * ct, axis=-1)
    dist = sq[:, :, None] + sq[:, None, :] - 2.0 * jnp.einsum('bnd,bmd->bnm', ct, ct)
    _, idx = jax.lax.top_k(-dist, _K)
    xt = x.transpose(0, 2, 1)
    nb = jax.vmap(lambda feat, ind: feat[ind])(xt, idx)
    nb = nb.transpose(0, 3, 1, 2)
    nb = nb - x[:, :, :, None]
    xq = x[:, :, :, None]
    q = jnp.einsum('oc,bcnk->bonk', Wq, xq)
    kk = jnp.einsum('oc,bcnk->bonk', Wk, nb)
    v = jnp.einsum('oc,bcnk->bonk', Wv, nb)

    def _split(t):
        b, _, n, kkk = t.shape
        t = t.reshape(b, _H, _DEPTH, n, kkk)
        return t.transpose(0, 1, 3, 4, 2)

    q = _split(q)
    kk = _split(kk).transpose(0, 1, 2, 4, 3)
    v = _split(v)
    energy = q @ kk
    attention = jax.nn.softmax(energy / math.sqrt(q.shape[-1]), axis=-1)
    out = (attention @ v)[:, :, :, 0, :]
    out = out.transpose(0, 2, 1, 3)
    out = out.reshape(out.shape[0], out.shape[1], -1).transpose(0, 2, 1)
    out = pl.pallas_call(
        _identity_kernel,
        out_shape=jax.ShapeDtypeStruct(out.shape, out.dtype),
    )(out)
    return out


# speedup vs baseline: 7.9182x; 7.9182x over previous
import functools
import math

import jax
import jax.numpy as jnp
import numpy as np
from jax.experimental import pallas as pl
from jax.experimental.pallas import tpu as pltpu
from jax.experimental.pallas import tpu_sc as plsc

B, N, K = 8, 2048, 32
CIN, COUT, H = 3, 64, 8
DEPTH = COUT // H
TILE = 256
ATT_TILE = 32
GATHER_WIN = 256


def _qkv_body(xt_ref, w_ref, q_ref, kv_ref):
    xt = xt_ref[...]
    w = w_ref[...]
    qkv = jnp.dot(xt, w, preferred_element_type=jnp.float32)
    q_ref[...] = qkv[:, :COUT]
    kv_ref[...] = qkv[:, COUT:]


def _project_qkv(xt_flat, w_cat):
    return pl.pallas_call(
        _qkv_body,
        out_shape=(
            jax.ShapeDtypeStruct((B * N, COUT), jnp.float32),
            jax.ShapeDtypeStruct((B * N, 2 * COUT), jnp.float32),
        ),
    )(xt_flat, w_cat)


def _knn_body(call_ref, ct_ref, ctile_ref, idx_ref):
    b = pl.program_id(0)
    ct_all = ct_ref[0]
    c_tile = ctile_ref[0]
    sq_all = jnp.sum(ct_all * ct_all, axis=1, keepdims=True)
    sq_t = jnp.sum(c_tile * c_tile, axis=0, keepdims=True)
    g = jnp.dot(ct_all, c_tile, preferred_element_type=jnp.float32)
    dist = sq_all + sq_t - 2.0 * g
    iota = jax.lax.broadcasted_iota(jnp.int32, (N, TILE), 0)
    big = jnp.float32(np.inf)
    for j in range(K):
        v = jnp.min(dist, axis=0, keepdims=True)
        cand = jnp.where(dist == v, iota, jnp.int32(N))
        pos = jnp.min(cand, axis=0, keepdims=True)
        idx_ref[0, j, :] = pos[0] + b * N
        dist = jnp.where(iota == pos, big, dist)
    del call_ref


def _knn_indices(coordinate, coord_t):
    grid = (B, N // TILE)
    return pl.pallas_call(
        _knn_body,
        grid=grid,
        in_specs=[
            pl.BlockSpec((1, CIN, N), lambda b, t: (b, 0, 0)),
            pl.BlockSpec((1, N, CIN), lambda b, t: (b, 0, 0)),
            pl.BlockSpec((1, CIN, TILE), lambda b, t: (b, 0, t)),
        ],
        out_specs=pl.BlockSpec((1, K, TILE), lambda b, t: (b, 0, t)),
        out_shape=jax.ShapeDtypeStruct((B, K, N), jnp.int32),
    )(coordinate, coord_t, coordinate)


def _sc_gather(kv_table, idx_flat):
    info = plsc.get_sparse_core_info()
    nw = info.num_cores * info.num_subcores
    total = idx_flat.shape[0]
    per_w = total // nw
    idx2d = idx_flat.reshape(1, total)
    mesh = plsc.VectorSubcoreMesh(core_axis_name="c", subcore_axis_name="s")

    @functools.partial(
        pl.kernel,
        mesh=mesh,
        out_type=jax.ShapeDtypeStruct((total, 2 * COUT), jnp.float32),
    )
    def gather_kernel(kv_hbm, i_hbm, o_hbm):
        def body(i_vmem, o_vmem):
            pltpu.sync_copy(kv_hbm.at[i_vmem.at[0]], o_vmem)

        pltpu.emit_pipeline(
            body,
            grid=(total // GATHER_WIN,),
            in_specs=[pl.BlockSpec((1, GATHER_WIN), index_map=lambda i: (0, i))],
            out_specs=[pl.BlockSpec((GATHER_WIN, 2 * COUT),
                                    index_map=lambda i: (i, 0))],
            core_axis_name=("c", "s"),
            dimension_semantics=(pltpu.PARALLEL,),
        )(i_hbm, o_hbm)

    del per_w
    return gather_kernel(kv_table, idx2d)


def _attn_body(q_ref, kvnb_ref, vself_ref, seg_ref, out_ref):
    q = q_ref[...]
    kvnb = kvnb_ref[...]
    knb = kvnb[:, :COUT].reshape(ATT_TILE, K, COUT)
    vnb = kvnb[:, COUT:].reshape(ATT_TILE, K, COUT)
    prod = knb * q[:, None, :]
    seg = seg_ref[...]
    e = jnp.dot(prod.reshape(ATT_TILE * K, COUT), seg,
                preferred_element_type=jnp.float32).reshape(ATT_TILE, K, COUT)
    m = jnp.max(e, axis=1, keepdims=True)
    ex = jnp.exp(e - m)
    s = jnp.sum(ex, axis=1, keepdims=True)
    a = ex / s
    out = jnp.sum(a * vnb, axis=1)
    out_ref[...] = out - vself_ref[...]


def _attention(q_table, kvnb, v_self):
    grid = (B * N // ATT_TILE,)
    seg = jnp.kron(jnp.eye(H, dtype=jnp.float32),
                   jnp.ones((DEPTH, DEPTH), jnp.float32)) / math.sqrt(DEPTH)
    return pl.pallas_call(
        _attn_body,
        grid=grid,
        in_specs=[
            pl.BlockSpec((ATT_TILE, COUT), lambda t: (t, 0)),
            pl.BlockSpec((ATT_TILE * K, 2 * COUT), lambda t: (t, 0)),
            pl.BlockSpec((ATT_TILE, COUT), lambda t: (t, 0)),
            pl.BlockSpec((COUT, COUT), lambda t: (0, 0)),
        ],
        out_specs=pl.BlockSpec((ATT_TILE, COUT), lambda t: (t, 0)),
        out_shape=jax.ShapeDtypeStruct((B * N, COUT), jnp.float32),
    )(q_table, kvnb, v_self, seg)


def kernel(x, coordinate, Wq, Wk, Wv):
    xt_flat = x.transpose(0, 2, 1).reshape(B * N, CIN)
    coord_t = coordinate.transpose(0, 2, 1)
    w_cat = jnp.concatenate([Wq.T, Wk.T, Wv.T], axis=1)

    q_table, kv_table = _project_qkv(xt_flat, w_cat)
    idx = _knn_indices(coordinate, coord_t)
    idx_flat = idx.transpose(0, 2, 1).reshape(B * N * K)
    kvnb = _sc_gather(kv_table, idx_flat)
    v_self = kv_table[:, COUT:]
    out_rows = _attention(q_table, kvnb, v_self)
    return out_rows.reshape(B, N, COUT).transpose(0, 2, 1)
